# baseline (speedup 1.0000x reference)
import jax
import jax.numpy as jnp
from jax import lax
from jax.experimental import pallas as pl
from jax.experimental.pallas import tpu as pltpu
from jax.experimental.pallas import tpu_sc as plsc

B = 16
N = 1000
L = 8000
LANES = 16
CHUNKS = L // LANES
NPAD = 1024
BOXPAD = 4096
GCH = 128
NGCH = NPAD // GCH


def _body(pt_hbm, boxes_hbm, scores_hbm, classes_hbm, counts_hbm,
          cols_sp, ids_v, idxl_v, idx5_v, idx6_v, idxb_v,
          boxes_v, scores_v, classes_f_v, classes_v, counts_v, sem, sem2):
    s = lax.axis_index("s")
    b = s

    @pl.when(s == 0)
    def _():
        pltpu.sync_copy(pt_hbm, cols_sp)

    plsc.subcore_barrier()

    cp = pltpu.async_copy(cols_sp.at[pl.ds(0, L)], ids_v, sem)

    zi = jnp.zeros((LANES,), jnp.int32)
    for j in range(NGCH):
        for k in range(GCH // LANES):
            idxl_v[j, pl.ds(k * LANES, LANES)] = zi

    cp.wait()

    bf = b.astype(jnp.float32)
    iota = lax.iota(jnp.int32, LANES)

    def step(i, off):
        base = i * LANES
        vb = ids_v[pl.ds(base, LANES)]
        mask = vb == bf
        incl = jnp.cumsum(jnp.where(mask, 1, 0).astype(jnp.int32))
        cnt = plsc.all_reduce_population_count(mask)
        ranks = off + incl - 1
        m2 = jnp.logical_and(mask, ranks < N)
        src = base + iota
        plsc.store_scatter(idxl_v, [ranks >> 7, ranks & 127], src, mask=m2)
        return off + cnt

    off = lax.fori_loop(0, CHUNKS, step, jnp.zeros((LANES,), jnp.int32))

    def mkidx(j2, carry):
        base = j2 * LANES
        r = idxl_v[j2 >> 3, pl.ds((j2 & 7) * LANES, LANES)]
        idx5_v[pl.ds(base, LANES)] = r + 5 * L
        idx6_v[pl.ds(base, LANES)] = r + 6 * L
        return carry

    lax.fori_loop(0, NPAD // LANES, mkidx, 0)

    comp1 = 1 + (iota & 3)
    subslot = iota >> 2

    def mkidxb(jj, carry):
        q0 = jj * LANES
        slots = (q0 >> 2) + subslot
        r = plsc.load_gather(idxl_v, [slots >> 7, slots & 127])
        idxb_v[pl.ds(q0, LANES)] = r + comp1 * L
        return carry

    lax.fori_loop(0, BOXPAD // LANES, mkidxb, 0)

    gathers = []
    for j in range(NGCH):
        gathers.append(pltpu.async_copy(
            cols_sp.at[idx5_v.at[pl.ds(j * GCH, GCH)]],
            scores_v.at[pl.ds(j * GCH, GCH)], sem2))
        gathers.append(pltpu.async_copy(
            cols_sp.at[idx6_v.at[pl.ds(j * GCH, GCH)]],
            classes_f_v.at[pl.ds(j * GCH, GCH)], sem2))
    for j in range(BOXPAD // GCH):
        gathers.append(pltpu.async_copy(
            cols_sp.at[idxb_v.at[pl.ds(j * GCH, GCH)]],
            boxes_v.at[pl.ds(j * GCH, GCH)], sem2))
    for g in gathers:
        g.wait()

    cnt_eff = jnp.minimum(off, N)
    zf = jnp.zeros((LANES,), jnp.float32)

    def post(j2, carry):
        base = j2 * LANES
        slots = base + iota
        mv = slots < cnt_eff
        scores_v[pl.ds(base, LANES)] = jnp.where(
            mv, scores_v[pl.ds(base, LANES)], zf)
        classes_v[pl.ds(base, LANES)] = jnp.where(
            mv, classes_f_v[pl.ds(base, LANES)], zf).astype(jnp.int32)
        return carry

    lax.fori_loop(0, NPAD // LANES, post, 0)

    def postb(jj, carry):
        q0 = jj * LANES
        bslot = (q0 >> 2) + subslot
        mb = bslot < cnt_eff
        boxes_v[pl.ds(q0, LANES)] = jnp.where(
            mb, boxes_v[pl.ds(q0, LANES)], zf)
        return carry

    lax.fori_loop(0, BOXPAD // LANES, postb, 0)

    for j in range(128 // LANES):
        counts_v[pl.ds(j * LANES, LANES)] = off

    pltpu.sync_copy(boxes_v, boxes_hbm.at[b])
    pltpu.sync_copy(scores_v, scores_hbm.at[b])
    pltpu.sync_copy(classes_v, classes_hbm.at[b])
    pltpu.sync_copy(counts_v, counts_hbm.at[b])


def kernel(predictions):
    pt = predictions.T.reshape(-1)
    mesh = plsc.VectorSubcoreMesh(
        core_axis_name="c", subcore_axis_name="s", num_cores=1)
    k = pl.kernel(
        _body,
        mesh=mesh,
        compiler_params=pltpu.CompilerParams(needs_layout_passes=False),
        out_type=[
            jax.ShapeDtypeStruct((B, BOXPAD), jnp.float32),
            jax.ShapeDtypeStruct((B, NPAD), jnp.float32),
            jax.ShapeDtypeStruct((B, NPAD), jnp.int32),
            jax.ShapeDtypeStruct((B, 128), jnp.int32),
        ],
        scratch_types=[
            pltpu.VMEM_SHARED((7 * L,), jnp.float32),
            pltpu.VMEM((L,), jnp.float32),
            pltpu.VMEM((NGCH, GCH), jnp.int32),
            pltpu.VMEM((NPAD,), jnp.int32),
            pltpu.VMEM((NPAD,), jnp.int32),
            pltpu.VMEM((BOXPAD,), jnp.int32),
            pltpu.VMEM((BOXPAD,), jnp.float32),
            pltpu.VMEM((NPAD,), jnp.float32),
            pltpu.VMEM((NPAD,), jnp.float32),
            pltpu.VMEM((NPAD,), jnp.int32),
            pltpu.VMEM((128,), jnp.int32),
            pltpu.SemaphoreType.DMA,
            pltpu.SemaphoreType.DMA,
        ],
    )
    boxes, scores, classes, counts = k(pt)
    num_predictions = counts[:, :1]
    pred_boxes = boxes[:, :4 * N].reshape(B, N, 4)
    return (num_predictions, pred_boxes, scores[:, :N], classes[:, :N])


# speedup vs baseline: 1.0010x; 1.0010x over previous
import jax
import jax.numpy as jnp
from jax import lax
from jax.experimental import pallas as pl
from jax.experimental.pallas import tpu as pltpu
from jax.experimental.pallas import tpu_sc as plsc

B = 16
N = 1000
L = 8000
LANES = 16
CHUNKS = L // LANES
NPAD = 1024
BOXPAD = 4096
GCH = 128
NGCH = NPAD // GCH


def _body(pt_hbm, boxes_hbm, scores_hbm, classes_hbm, counts_hbm,
          cols_sp, ids_v, idxl_v, idx5_v, idx6_v, idxb_v,
          boxes_v, scores_v, classes_f_v, classes_v, counts_v, sem, sem2):
    s = lax.axis_index("s")
    b = s

    @pl.when(s == 0)
    def _():
        pltpu.sync_copy(pt_hbm, cols_sp)

    plsc.subcore_barrier()

    cp = pltpu.async_copy(cols_sp.at[pl.ds(0, L)], ids_v, sem)

    zi = jnp.zeros((LANES,), jnp.int32)
    for j in range(NGCH):
        for k in range(GCH // LANES):
            idxl_v[j, pl.ds(k * LANES, LANES)] = zi

    cp.wait()

    bf = b.astype(jnp.float32)
    iota = lax.iota(jnp.int32, LANES)

    def step(i, off):
        base = i * LANES
        vb = ids_v[pl.ds(base, LANES)]
        mask = vb == bf
        incl = jnp.cumsum(jnp.where(mask, 1, 0).astype(jnp.int32))
        cnt = plsc.all_reduce_population_count(mask)
        ranks = off + incl - 1
        m2 = jnp.logical_and(mask, ranks < N)
        src = base + iota
        plsc.store_scatter(idxl_v, [ranks >> 7, ranks & 127], src, mask=m2)
        return off + cnt

    _s1 = jax.named_scope("ph_scan"); _s1.__enter__()
    off = lax.fori_loop(0, CHUNKS, step, jnp.zeros((LANES,), jnp.int32))
    _s1.__exit__(None, None, None)
    _s2 = jax.named_scope("ph_mkidx"); _s2.__enter__()

    def mkidx(j2, carry):
        base = j2 * LANES
        r = idxl_v[j2 >> 3, pl.ds((j2 & 7) * LANES, LANES)]
        idx5_v[pl.ds(base, LANES)] = r + 5 * L
        idx6_v[pl.ds(base, LANES)] = r + 6 * L
        return carry

    lax.fori_loop(0, NPAD // LANES, mkidx, 0)

    comp1 = 1 + (iota & 3)
    subslot = iota >> 2

    def mkidxb(jj, carry):
        q0 = jj * LANES
        slots = (q0 >> 2) + subslot
        r = plsc.load_gather(idxl_v, [slots >> 7, slots & 127])
        idxb_v[pl.ds(q0, LANES)] = r + comp1 * L
        return carry

    lax.fori_loop(0, BOXPAD // LANES, mkidxb, 0)
    _s2.__exit__(None, None, None)
    _s3 = jax.named_scope("ph_gather"); _s3.__enter__()

    gathers = []
    for j in range(NGCH):
        gathers.append(pltpu.async_copy(
            cols_sp.at[idx5_v.at[pl.ds(j * GCH, GCH)]],
            scores_v.at[pl.ds(j * GCH, GCH)], sem2))
        gathers.append(pltpu.async_copy(
            cols_sp.at[idx6_v.at[pl.ds(j * GCH, GCH)]],
            classes_f_v.at[pl.ds(j * GCH, GCH)], sem2))
    for j in range(BOXPAD // GCH):
        gathers.append(pltpu.async_copy(
            cols_sp.at[idxb_v.at[pl.ds(j * GCH, GCH)]],
            boxes_v.at[pl.ds(j * GCH, GCH)], sem2))
    for g in gathers:
        g.wait()
    _s3.__exit__(None, None, None)
    _s4 = jax.named_scope("ph_post"); _s4.__enter__()

    cnt_eff = jnp.minimum(off, N)
    zf = jnp.zeros((LANES,), jnp.float32)

    def post(j2, carry):
        base = j2 * LANES
        slots = base + iota
        mv = slots < cnt_eff
        scores_v[pl.ds(base, LANES)] = jnp.where(
            mv, scores_v[pl.ds(base, LANES)], zf)
        classes_v[pl.ds(base, LANES)] = jnp.where(
            mv, classes_f_v[pl.ds(base, LANES)], zf).astype(jnp.int32)
        return carry

    lax.fori_loop(0, NPAD // LANES, post, 0)

    def postb(jj, carry):
        q0 = jj * LANES
        bslot = (q0 >> 2) + subslot
        mb = bslot < cnt_eff
        boxes_v[pl.ds(q0, LANES)] = jnp.where(
            mb, boxes_v[pl.ds(q0, LANES)], zf)
        return carry

    lax.fori_loop(0, BOXPAD // LANES, postb, 0)
    _s4.__exit__(None, None, None)

    for j in range(128 // LANES):
        counts_v[pl.ds(j * LANES, LANES)] = off

    pltpu.sync_copy(boxes_v, boxes_hbm.at[b])
    pltpu.sync_copy(scores_v, scores_hbm.at[b])
    pltpu.sync_copy(classes_v, classes_hbm.at[b])
    pltpu.sync_copy(counts_v, counts_hbm.at[b])


def kernel(predictions):
    pt = predictions.T.reshape(-1)
    mesh = plsc.VectorSubcoreMesh(
        core_axis_name="c", subcore_axis_name="s", num_cores=1)
    k = pl.kernel(
        _body,
        mesh=mesh,
        compiler_params=pltpu.CompilerParams(needs_layout_passes=False),
        out_type=[
            jax.ShapeDtypeStruct((B, BOXPAD), jnp.float32),
            jax.ShapeDtypeStruct((B, NPAD), jnp.float32),
            jax.ShapeDtypeStruct((B, NPAD), jnp.int32),
            jax.ShapeDtypeStruct((B, 128), jnp.int32),
        ],
        scratch_types=[
            pltpu.VMEM_SHARED((7 * L,), jnp.float32),
            pltpu.VMEM((L,), jnp.float32),
            pltpu.VMEM((NGCH, GCH), jnp.int32),
            pltpu.VMEM((NPAD,), jnp.int32),
            pltpu.VMEM((NPAD,), jnp.int32),
            pltpu.VMEM((BOXPAD,), jnp.int32),
            pltpu.VMEM((BOXPAD,), jnp.float32),
            pltpu.VMEM((NPAD,), jnp.float32),
            pltpu.VMEM((NPAD,), jnp.float32),
            pltpu.VMEM((NPAD,), jnp.int32),
            pltpu.VMEM((128,), jnp.int32),
            pltpu.SemaphoreType.DMA,
            pltpu.SemaphoreType.DMA,
        ],
    )
    boxes, scores, classes, counts = k(pt)
    num_predictions = counts[:, :1]
    pred_boxes = boxes[:, :4 * N].reshape(B, N, 4)
    return (num_predictions, pred_boxes, scores[:, :N], classes[:, :N])
